# baseline (speedup 1.0000x reference)
import functools

import jax
import jax.numpy as jnp
from jax import lax
from jax.experimental import pallas as pl
from jax.experimental.pallas import tpu as pltpu
from jax.experimental.pallas import tpu_sc as plsc

NC = 2
NS = 16
NW = NC * NS
CH = 128

_f32 = jnp.float32

_SC_PARAMS = pltpu.CompilerParams(
    needs_layout_passes=False, use_tc_tiling_on_sc=False)


def _degrees(srcT, dstT, n_pad):
    ns_, nct, ch = srcT.shape
    nh = nct // NC
    mesh = plsc.VectorSubcoreMesh(core_axis_name="c", subcore_axis_name="s")

    @functools.partial(
        pl.kernel,
        out_type=(
            jax.ShapeDtypeStruct((NW, n_pad), _f32),
            jax.ShapeDtypeStruct((NW, n_pad), _f32),
        ),
        mesh=mesh,
        compiler_params=_SC_PARAMS,
        scratch_types=[
            pltpu.VMEM((nh, ch), jnp.int32),
            pltpu.VMEM((nh, ch), jnp.int32),
            pltpu.VMEM((n_pad,), _f32),
            pltpu.VMEM((n_pad,), _f32),
        ],
    )
    def deg_kernel(src_hbm, dst_hbm, out_s, out_d, src_v, dst_v, acc_s, acc_d):
        c = lax.axis_index("c")
        s = lax.axis_index("s")
        wid = c * NS + s
        pltpu.sync_copy(src_hbm.at[s, pl.ds(c * nh, nh)], src_v)
        pltpu.sync_copy(dst_hbm.at[s, pl.ds(c * nh, nh)], dst_v)
        zeros16 = jnp.zeros((16,), _f32)

        def zbody(i, carry):
            acc_s[pl.ds(i * 16, 16)] = zeros16
            acc_d[pl.ds(i * 16, 16)] = zeros16
            return carry

        lax.fori_loop(0, n_pad // 16, zbody, 0)
        ones16 = jnp.ones((16,), _f32)

        def cbody(i, carry):
            for j in range(ch // 16):
                si = src_v[i, pl.ds(j * 16, 16)]
                plsc.addupdate_scatter(acc_s, [si], ones16)
                di = dst_v[i, pl.ds(j * 16, 16)]
                plsc.addupdate_scatter(acc_d, [di], ones16)
            return carry

        lax.fori_loop(0, nh, cbody, 0)
        pltpu.sync_copy(acc_s, out_s.at[wid])
        pltpu.sync_copy(acc_d, out_d.at[wid])

    return deg_kernel(srcT, dstT)


def _aggregate(ta, tb, srcT, dstT, n_pad):
    npad_, dh = ta.shape
    ns_, nct, ch = srcT.shape
    nhalf = nct // 2
    rps = n_pad // NS
    mesh = plsc.VectorSubcoreMesh(core_axis_name="c", subcore_axis_name="s")

    @functools.partial(
        pl.kernel,
        out_type=jax.ShapeDtypeStruct((NC, n_pad, dh), _f32),
        mesh=mesh,
        compiler_params=_SC_PARAMS,
        scratch_types=[
            pltpu.VMEM((nhalf, ch), jnp.int32),
            pltpu.VMEM((nhalf, ch), jnp.int32),
            pltpu.VMEM((ch, dh), _f32),
            pltpu.VMEM_SHARED((n_pad, dh), _f32),
            pltpu.VMEM_SHARED((n_pad, dh), _f32),
            pltpu.SemaphoreType.DMA,
        ],
    )
    def agg_kernel(ta_hbm, tb_hbm, src_hbm, dst_hbm, out_hbm,
                   src_v, dst_v, rows_v, sp_t, sp_a, sem):
        c = lax.axis_index("c")
        s = lax.axis_index("s")
        row0 = s * rps
        @pl.when(c == 0)
        def _():
            pltpu.sync_copy(ta_hbm.at[pl.ds(row0, rps)],
                            sp_t.at[pl.ds(row0, rps)])

        @pl.when(c == 1)
        def _():
            pltpu.sync_copy(tb_hbm.at[pl.ds(row0, rps)],
                            sp_t.at[pl.ds(row0, rps)])

        zeros16 = jnp.zeros((16,), _f32)
        zoffs = list(range(0, dh - 15, 16))
        if dh % 16:
            zoffs.append(dh - 16)

        def zbody(i, carry):
            for o in zoffs:
                rows_v[i, pl.ds(o, 16)] = zeros16
            return carry

        lax.fori_loop(0, ch, zbody, 0)
        nfull = rps // ch

        def zcopy(k, carry):
            pltpu.sync_copy(rows_v, sp_a.at[pl.ds(row0 + k * ch, ch)])
            return carry

        lax.fori_loop(0, nfull, zcopy, 0)
        rem = rps - nfull * ch
        if rem:
            pltpu.sync_copy(rows_v.at[pl.ds(0, rem)],
                            sp_a.at[pl.ds(row0 + nfull * ch, rem)])
        plsc.subcore_barrier()

        def run_half(h):
            pltpu.sync_copy(src_hbm.at[s, pl.ds(h * nhalf, nhalf)], src_v)
            pltpu.sync_copy(dst_hbm.at[s, pl.ds(h * nhalf, nhalf)], dst_v)

            def body(i, carry):
                pltpu.async_copy(sp_t.at[src_v.at[i]], rows_v, sem).wait()
                pltpu.sync_copy(rows_v, sp_a.at[dst_v.at[i]], add=True)
                return carry

            lax.fori_loop(0, nhalf, body, 0)

        run_half(0)
        run_half(1)
        plsc.subcore_barrier()
        pltpu.sync_copy(sp_a.at[pl.ds(row0, rps)],
                        out_hbm.at[c, pl.ds(row0, rps)])

    return agg_kernel(ta, tb, srcT, dstT)


def _norms_scale(deg_s, deg_d, ua, ub, n_pad):
    nw = deg_s.shape[0]
    dh = ua.shape[1]

    def body(ds_ref, dd_ref, ua_ref, ub_ref,
             ns_ref, nd_ref, ta_ref, tb_ref):
        ones = jnp.ones((nw, 1), _f32)
        dn = (((0,), (0,)), ((), ()))
        ssum = lax.dot_general(ds_ref[...], ones, dn,
                               preferred_element_type=_f32)
        dsum = lax.dot_general(dd_ref[...], ones, dn,
                               preferred_element_type=_f32)
        ns = lax.rsqrt(jnp.maximum(ssum, 1.0))
        ns_ref[...] = ns
        nd_ref[...] = lax.rsqrt(jnp.maximum(dsum, 1.0))
        ta_ref[...] = ua_ref[...] * ns
        tb_ref[...] = ub_ref[...] * ns

    return pl.pallas_call(
        body,
        out_shape=[
            jax.ShapeDtypeStruct((n_pad, 1), _f32),
            jax.ShapeDtypeStruct((n_pad, 1), _f32),
            jax.ShapeDtypeStruct((n_pad, dh), _f32),
            jax.ShapeDtypeStruct((n_pad, dh), _f32),
        ],
    )(deg_s, deg_d, ua, ub)


def _row_block(n):
    for b in (1264, 2000, 1000, 500, 250, 200, 100, 50, 25, 8):
        if n % b == 0:
            return b
    return n


def _matmul(x, wa, wb, n_pad):
    rb = _row_block(n_pad)
    d_in = x.shape[1]
    dh = wa.shape[1]

    def body(x_ref, wa_ref, wb_ref, oa_ref, ob_ref):
        xb = x_ref[...]
        oa_ref[...] = jnp.dot(xb, wa_ref[...], preferred_element_type=_f32)
        ob_ref[...] = jnp.dot(xb, wb_ref[...], preferred_element_type=_f32)

    return pl.pallas_call(
        body,
        grid=(n_pad // rb,),
        in_specs=[
            pl.BlockSpec((rb, d_in), lambda i: (i, 0)),
            pl.BlockSpec((d_in, dh), lambda i: (0, 0)),
            pl.BlockSpec((d_in, dh), lambda i: (0, 0)),
        ],
        out_specs=[
            pl.BlockSpec((rb, dh), lambda i: (i, 0)),
            pl.BlockSpec((rb, dh), lambda i: (i, 0)),
        ],
        out_shape=[
            jax.ShapeDtypeStruct((n_pad, dh), _f32),
            jax.ShapeDtypeStruct((n_pad, dh), _f32),
        ],
    )(x, wa, wb)


def _mid(p, nd, b1, ns, w2a, w2b, n_pad):
    rb = _row_block(n_pad)
    nc, _, dh = p.shape
    do = w2a.shape[1]

    def body(p_ref, nd_ref, b_ref, ns_ref, wa_ref, wb_ref, oa_ref, ob_ref):
        h = jnp.concatenate([p_ref[0], p_ref[1]], axis=1)
        h = jnp.maximum(h * nd_ref[...] + b_ref[...], 0.0)
        hs = h * ns_ref[...]
        oa_ref[...] = jnp.dot(hs, wa_ref[...], preferred_element_type=_f32)
        ob_ref[...] = jnp.dot(hs, wb_ref[...], preferred_element_type=_f32)

    return pl.pallas_call(
        body,
        grid=(n_pad // rb,),
        in_specs=[
            pl.BlockSpec((nc, rb, dh), lambda i: (0, i, 0)),
            pl.BlockSpec((rb, 1), lambda i: (i, 0)),
            pl.BlockSpec((1, 2 * dh), lambda i: (0, 0)),
            pl.BlockSpec((rb, 1), lambda i: (i, 0)),
            pl.BlockSpec((2 * dh, do), lambda i: (0, 0)),
            pl.BlockSpec((2 * dh, do), lambda i: (0, 0)),
        ],
        out_specs=[
            pl.BlockSpec((rb, do), lambda i: (i, 0)),
            pl.BlockSpec((rb, do), lambda i: (i, 0)),
        ],
        out_shape=[
            jax.ShapeDtypeStruct((n_pad, do), _f32),
            jax.ShapeDtypeStruct((n_pad, do), _f32),
        ],
    )(p, nd, b1, ns, w2a, w2b)


def _final(p, nd, b2, n):
    rb = _row_block(n)
    nc, _, dh = p.shape

    def body(p_ref, nd_ref, b_ref, o_ref):
        o = jnp.concatenate([p_ref[0], p_ref[1]], axis=1)
        o_ref[...] = o * nd_ref[...] + b_ref[...]

    return pl.pallas_call(
        body,
        grid=(n // rb,),
        in_specs=[
            pl.BlockSpec((nc, rb, dh), lambda i: (0, i, 0)),
            pl.BlockSpec((rb, 1), lambda i: (i, 0)),
            pl.BlockSpec((1, 2 * dh), lambda i: (0, 0)),
        ],
        out_specs=pl.BlockSpec((rb, 2 * dh), lambda i: (i, 0)),
        out_shape=jax.ShapeDtypeStruct((n, 2 * dh), _f32),
    )(p, nd, b2)


def kernel(x, edge_index, W1, b1, W2, b2):
    n, d_in = x.shape
    hid = W1.shape[1]
    ncls = W2.shape[1]
    e = edge_index.shape[1]
    nct = -(-e // (NS * CH))
    nct = ((nct + 1) // 2) * 2
    e_pad = nct * NS * CH
    n_pad = ((n + 1 + 127) // 128) * 128

    src = edge_index[0]
    dst = edge_index[1]
    trash = jnp.full((e_pad - e,), n, jnp.int32)
    srcT = jnp.concatenate([src, trash]).reshape(NS, nct, CH)
    dstT = jnp.concatenate([dst, trash]).reshape(NS, nct, CH)

    x_pad = jnp.concatenate([x, jnp.zeros((n_pad - n, d_in), _f32)])
    h1 = hid // 2
    deg_s, deg_d = _degrees(srcT, dstT, n_pad)
    u1a, u1b = _matmul(x_pad, W1[:, :h1], W1[:, h1:], n_pad)
    ns, nd, t1a, t1b = _norms_scale(deg_s, deg_d, u1a, u1b, n_pad)
    p1 = _aggregate(t1a, t1b, srcT, dstT, n_pad)

    h2 = ((-(-ncls // 2)) + 7) // 8 * 8
    ncp = 2 * h2
    W2p = jnp.concatenate([W2, jnp.zeros((hid, ncp - ncls), _f32)], axis=1)
    b2p = jnp.concatenate([b2, jnp.zeros((ncp - ncls,), _f32)])
    t2a, t2b = _mid(p1, nd, b1.reshape(1, hid), ns,
                    W2p[:, :h2], W2p[:, h2:], n_pad)
    p2 = _aggregate(t2a, t2b, srcT, dstT, n_pad)

    out = _final(p2, nd, b2p.reshape(1, ncp), n)
    return out[:, :ncls]


# speedup vs baseline: 1.0548x; 1.0548x over previous
import functools

import jax
import jax.numpy as jnp
from jax import lax
from jax.experimental import pallas as pl
from jax.experimental.pallas import tpu as pltpu
from jax.experimental.pallas import tpu_sc as plsc

NC = 2
NS = 16
NW = NC * NS
CH = 128

_f32 = jnp.float32

_SC_PARAMS = pltpu.CompilerParams(
    needs_layout_passes=False, use_tc_tiling_on_sc=False)


def _degrees(srcT, dstT, n_pad):
    ns_, nct, ch = srcT.shape
    nh = nct // NC
    mesh = plsc.VectorSubcoreMesh(core_axis_name="c", subcore_axis_name="s")

    @functools.partial(
        pl.kernel,
        out_type=(
            jax.ShapeDtypeStruct((NW, n_pad), _f32),
            jax.ShapeDtypeStruct((NW, n_pad), _f32),
        ),
        mesh=mesh,
        compiler_params=_SC_PARAMS,
        scratch_types=[
            pltpu.VMEM((nh, ch), jnp.int32),
            pltpu.VMEM((nh, ch), jnp.int32),
            pltpu.VMEM((n_pad,), _f32),
            pltpu.VMEM((n_pad,), _f32),
        ],
    )
    def deg_kernel(src_hbm, dst_hbm, out_s, out_d, src_v, dst_v, acc_s, acc_d):
        c = lax.axis_index("c")
        s = lax.axis_index("s")
        wid = c * NS + s
        pltpu.sync_copy(src_hbm.at[s, pl.ds(c * nh, nh)], src_v)
        pltpu.sync_copy(dst_hbm.at[s, pl.ds(c * nh, nh)], dst_v)
        zeros16 = jnp.zeros((16,), _f32)

        def zbody(i, carry):
            acc_s[pl.ds(i * 16, 16)] = zeros16
            acc_d[pl.ds(i * 16, 16)] = zeros16
            return carry

        lax.fori_loop(0, n_pad // 16, zbody, 0)
        ones16 = jnp.ones((16,), _f32)

        def cbody(i, carry):
            for j in range(ch // 16):
                si = src_v[i, pl.ds(j * 16, 16)]
                plsc.addupdate_scatter(acc_s, [si], ones16)
                di = dst_v[i, pl.ds(j * 16, 16)]
                plsc.addupdate_scatter(acc_d, [di], ones16)
            return carry

        lax.fori_loop(0, nh, cbody, 0)
        pltpu.sync_copy(acc_s, out_s.at[wid])
        pltpu.sync_copy(acc_d, out_d.at[wid])

    return deg_kernel(srcT, dstT)


def _aggregate(ta, tb, srcT, dstT, n_pad):
    npad_, dh = ta.shape
    ns_, nct, ch = srcT.shape
    nhalf = nct // 2
    rps = n_pad // NS
    mesh = plsc.VectorSubcoreMesh(core_axis_name="c", subcore_axis_name="s")

    @functools.partial(
        pl.kernel,
        out_type=jax.ShapeDtypeStruct((NC, n_pad, dh), _f32),
        mesh=mesh,
        compiler_params=_SC_PARAMS,
        scratch_types=[
            pltpu.VMEM((nhalf, ch), jnp.int32),
            pltpu.VMEM((nhalf, ch), jnp.int32),
            pltpu.VMEM((ch, dh), _f32),
            pltpu.VMEM_SHARED((n_pad, dh), _f32),
            pltpu.VMEM_SHARED((n_pad, dh), _f32),
            pltpu.SemaphoreType.DMA,
        ],
    )
    def agg_kernel(ta_hbm, tb_hbm, src_hbm, dst_hbm, out_hbm,
                   src_v, dst_v, rows_v, sp_t, sp_a, sem):
        c = lax.axis_index("c")
        s = lax.axis_index("s")
        row0 = s * rps
        @pl.when(c == 0)
        def _():
            pltpu.sync_copy(ta_hbm.at[pl.ds(row0, rps)],
                            sp_t.at[pl.ds(row0, rps)])

        @pl.when(c == 1)
        def _():
            pltpu.sync_copy(tb_hbm.at[pl.ds(row0, rps)],
                            sp_t.at[pl.ds(row0, rps)])

        zeros16 = jnp.zeros((16,), _f32)
        zoffs = list(range(0, dh - 15, 16))
        if dh % 16:
            zoffs.append(dh - 16)

        def zbody(i, carry):
            for o in zoffs:
                rows_v[i, pl.ds(o, 16)] = zeros16
            return carry

        lax.fori_loop(0, ch, zbody, 0)
        nfull = rps // ch

        def zcopy(k, carry):
            pltpu.sync_copy(rows_v, sp_a.at[pl.ds(row0 + k * ch, ch)])
            return carry

        lax.fori_loop(0, nfull, zcopy, 0)
        rem = rps - nfull * ch
        if rem:
            pltpu.sync_copy(rows_v.at[pl.ds(0, rem)],
                            sp_a.at[pl.ds(row0 + nfull * ch, rem)])
        plsc.subcore_barrier()

        def run_half(h):
            pltpu.sync_copy(src_hbm.at[s, pl.ds(h * nhalf, nhalf)], src_v)
            pltpu.sync_copy(dst_hbm.at[s, pl.ds(h * nhalf, nhalf)], dst_v)

            def body(i, carry):
                pltpu.async_copy(sp_t.at[src_v.at[i]], rows_v, sem).wait()
                pltpu.sync_copy(rows_v, sp_a.at[dst_v.at[i]], add=True)
                return carry

            lax.fori_loop(0, nhalf, body, 0)

        run_half(0)
        run_half(1)
        plsc.subcore_barrier()
        pltpu.sync_copy(sp_a.at[pl.ds(row0, rps)],
                        out_hbm.at[c, pl.ds(row0, rps)])

    return agg_kernel(ta, tb, srcT, dstT)


def _aggregate_es(t, srcT, dstT, n_pad):
    npad_, dh = t.shape
    ns_, nct, ch = srcT.shape
    nh = nct // 2
    rps = n_pad // NS
    mesh = plsc.VectorSubcoreMesh(core_axis_name="c", subcore_axis_name="s")

    @functools.partial(
        pl.kernel,
        out_type=jax.ShapeDtypeStruct((NC, n_pad, dh), _f32),
        mesh=mesh,
        compiler_params=_SC_PARAMS,
        scratch_types=[
            pltpu.VMEM((nh, ch), jnp.int32),
            pltpu.VMEM((nh, ch), jnp.int32),
            pltpu.VMEM((ch, dh), _f32),
            pltpu.VMEM_SHARED((n_pad, dh), _f32),
            pltpu.VMEM_SHARED((n_pad, dh), _f32),
            pltpu.SemaphoreType.DMA,
        ],
    )
    def agg_kernel(t_hbm, src_hbm, dst_hbm, out_hbm,
                   src_v, dst_v, rows_v, sp_t, sp_a, sem):
        c = lax.axis_index("c")
        s = lax.axis_index("s")
        row0 = s * rps
        pltpu.sync_copy(t_hbm.at[pl.ds(row0, rps)], sp_t.at[pl.ds(row0, rps)])
        pltpu.sync_copy(src_hbm.at[s, pl.ds(c * nh, nh)], src_v)
        pltpu.sync_copy(dst_hbm.at[s, pl.ds(c * nh, nh)], dst_v)

        zeros16 = jnp.zeros((16,), _f32)
        zoffs = list(range(0, dh - 15, 16))
        if dh % 16:
            zoffs.append(dh - 16)

        def zbody(i, carry):
            for o in zoffs:
                rows_v[i, pl.ds(o, 16)] = zeros16
            return carry

        lax.fori_loop(0, ch, zbody, 0)
        nfull = rps // ch

        def zcopy(k, carry):
            pltpu.sync_copy(rows_v, sp_a.at[pl.ds(row0 + k * ch, ch)])
            return carry

        lax.fori_loop(0, nfull, zcopy, 0)
        rem = rps - nfull * ch
        if rem:
            pltpu.sync_copy(rows_v.at[pl.ds(0, rem)],
                            sp_a.at[pl.ds(row0 + nfull * ch, rem)])
        plsc.subcore_barrier()

        def body(i, carry):
            pltpu.async_copy(sp_t.at[src_v.at[i]], rows_v, sem).wait()
            pltpu.sync_copy(rows_v, sp_a.at[dst_v.at[i]], add=True)
            return carry

        lax.fori_loop(0, nh, body, 0)
        plsc.subcore_barrier()
        pltpu.sync_copy(sp_a.at[pl.ds(row0, rps)],
                        out_hbm.at[c, pl.ds(row0, rps)])

    return agg_kernel(t, srcT, dstT)


def _norms_scale(deg_s, deg_d, ua, ub, n_pad):
    nw = deg_s.shape[0]
    dh = ua.shape[1]

    def body(ds_ref, dd_ref, ua_ref, ub_ref,
             ns_ref, nd_ref, ta_ref, tb_ref):
        ones = jnp.ones((nw, 1), _f32)
        dn = (((0,), (0,)), ((), ()))
        ssum = lax.dot_general(ds_ref[...], ones, dn,
                               preferred_element_type=_f32)
        dsum = lax.dot_general(dd_ref[...], ones, dn,
                               preferred_element_type=_f32)
        ns = lax.rsqrt(jnp.maximum(ssum, 1.0))
        ns_ref[...] = ns
        nd_ref[...] = lax.rsqrt(jnp.maximum(dsum, 1.0))
        ta_ref[...] = ua_ref[...] * ns
        tb_ref[...] = ub_ref[...] * ns

    return pl.pallas_call(
        body,
        out_shape=[
            jax.ShapeDtypeStruct((n_pad, 1), _f32),
            jax.ShapeDtypeStruct((n_pad, 1), _f32),
            jax.ShapeDtypeStruct((n_pad, dh), _f32),
            jax.ShapeDtypeStruct((n_pad, dh), _f32),
        ],
    )(deg_s, deg_d, ua, ub)


def _row_block(n):
    for b in (1264, 2000, 1000, 500, 250, 200, 100, 50, 25, 8):
        if n % b == 0:
            return b
    return n


def _matmul(x, wa, wb, n_pad):
    rb = _row_block(n_pad)
    d_in = x.shape[1]
    dh = wa.shape[1]

    def body(x_ref, wa_ref, wb_ref, oa_ref, ob_ref):
        xb = x_ref[...]
        oa_ref[...] = jnp.dot(xb, wa_ref[...], preferred_element_type=_f32)
        ob_ref[...] = jnp.dot(xb, wb_ref[...], preferred_element_type=_f32)

    return pl.pallas_call(
        body,
        grid=(n_pad // rb,),
        in_specs=[
            pl.BlockSpec((rb, d_in), lambda i: (i, 0)),
            pl.BlockSpec((d_in, dh), lambda i: (0, 0)),
            pl.BlockSpec((d_in, dh), lambda i: (0, 0)),
        ],
        out_specs=[
            pl.BlockSpec((rb, dh), lambda i: (i, 0)),
            pl.BlockSpec((rb, dh), lambda i: (i, 0)),
        ],
        out_shape=[
            jax.ShapeDtypeStruct((n_pad, dh), _f32),
            jax.ShapeDtypeStruct((n_pad, dh), _f32),
        ],
    )(x, wa, wb)


def _mid(p, nd, b1, ns, w2, n_pad):
    rb = _row_block(n_pad)
    nc, _, dh = p.shape
    do = w2.shape[1]

    def body(p_ref, nd_ref, b_ref, ns_ref, w_ref, o_ref):
        h = jnp.concatenate([p_ref[0], p_ref[1]], axis=1)
        h = jnp.maximum(h * nd_ref[...] + b_ref[...], 0.0)
        hs = h * ns_ref[...]
        o_ref[...] = jnp.dot(hs, w_ref[...], preferred_element_type=_f32)

    return pl.pallas_call(
        body,
        grid=(n_pad // rb,),
        in_specs=[
            pl.BlockSpec((nc, rb, dh), lambda i: (0, i, 0)),
            pl.BlockSpec((rb, 1), lambda i: (i, 0)),
            pl.BlockSpec((1, 2 * dh), lambda i: (0, 0)),
            pl.BlockSpec((rb, 1), lambda i: (i, 0)),
            pl.BlockSpec((2 * dh, do), lambda i: (0, 0)),
        ],
        out_specs=pl.BlockSpec((rb, do), lambda i: (i, 0)),
        out_shape=jax.ShapeDtypeStruct((n_pad, do), _f32),
    )(p, nd, b1, ns, w2)


def _final(p, nd, b2, n):
    rb = _row_block(n)
    nc, _, dh = p.shape

    def body(p_ref, nd_ref, b_ref, o_ref):
        o = p_ref[0] + p_ref[1]
        o_ref[...] = o * nd_ref[...] + b_ref[...]

    return pl.pallas_call(
        body,
        grid=(n // rb,),
        in_specs=[
            pl.BlockSpec((nc, rb, dh), lambda i: (0, i, 0)),
            pl.BlockSpec((rb, 1), lambda i: (i, 0)),
            pl.BlockSpec((1, dh), lambda i: (0, 0)),
        ],
        out_specs=pl.BlockSpec((rb, dh), lambda i: (i, 0)),
        out_shape=jax.ShapeDtypeStruct((n, dh), _f32),
    )(p, nd, b2)


def kernel(x, edge_index, W1, b1, W2, b2):
    n, d_in = x.shape
    hid = W1.shape[1]
    ncls = W2.shape[1]
    e = edge_index.shape[1]
    nct = -(-e // (NS * CH))
    nct = ((nct + 1) // 2) * 2
    e_pad = nct * NS * CH
    n_pad = ((n + 1 + 127) // 128) * 128

    src = edge_index[0]
    dst = edge_index[1]
    trash = jnp.full((e_pad - e,), n, jnp.int32)
    srcT = jnp.concatenate([src, trash]).reshape(NS, nct, CH)
    dstT = jnp.concatenate([dst, trash]).reshape(NS, nct, CH)

    x_pad = jnp.concatenate([x, jnp.zeros((n_pad - n, d_in), _f32)])
    h1 = hid // 2
    deg_s, deg_d = _degrees(srcT, dstT, n_pad)
    u1a, u1b = _matmul(x_pad, W1[:, :h1], W1[:, h1:], n_pad)
    ns, nd, t1a, t1b = _norms_scale(deg_s, deg_d, u1a, u1b, n_pad)
    p1 = _aggregate(t1a, t1b, srcT, dstT, n_pad)

    t2 = _mid(p1, nd, b1.reshape(1, hid), ns, W2, n_pad)
    p2 = _aggregate_es(t2, srcT, dstT, n_pad)

    return _final(p2, nd, b2.reshape(1, ncls), n)
